# Initial kernel scaffold; baseline (speedup 1.0000x reference)
import jax
import jax.numpy as jnp
from jax.experimental import pallas as pl


def kernel(x, deg, edge_idx, edge_attr, W_pre0, b_pre0, W_pre1, b_pre1, emb_table, W_gate, W_value, W_post, b_post, degree_param):
    raise NotImplementedError("write your pallas kernel here")


# speedup vs baseline: 99.6799x; 99.6799x over previous
import functools

import jax
import jax.numpy as jnp
import numpy as np
from jax import lax
from jax.experimental import pallas as pl
from jax.experimental.pallas import tpu as pltpu
from jax.experimental.pallas import tpu_sc as plsc

N = 10000
E = 320000
WIDTH = 128
C1 = 256
CA = 256
L = 4
BOND = 64
NUM_HEAD = 8
EPS = 1e-5

CH = 128
NCH = E // CH
NW = 32
TRIPS = -(-NCH // NW)
EB = 2000
NP = 10240
ROWS_PT = NP // 16

_mesh = plsc.VectorSubcoreMesh(core_axis_name="c", subcore_axis_name="s")


@functools.partial(
    pl.kernel,
    out_type=(jax.ShapeDtypeStruct((E, WIDTH), jnp.float32),
              jax.ShapeDtypeStruct((E, WIDTH), jnp.float32)),
    mesh=_mesh,
    scratch_types=[
        pltpu.VMEM((CH,), jnp.int32),
        pltpu.VMEM((CH,), jnp.int32),
        pltpu.VMEM((CH, WIDTH), jnp.float32),
        pltpu.VMEM((CH, WIDTH), jnp.float32),
        pltpu.SemaphoreType.DMA,
        pltpu.SemaphoreType.DMA,
    ],
)
def _sc_gather(src_hbm, dst_hbm, x_hbm, xs_out, xd_out,
               idx_s, idx_d, rows_s, rows_d, sem_s, sem_d):
    i32 = jnp.int32
    cid = lax.axis_index("c")
    sid = lax.axis_index("s")
    wid = sid * i32(2) + cid

    def body(k, carry):
        c = wid + k * i32(NW)

        @pl.when(c < i32(NCH))
        def _go():
            base = c * i32(CH)
            pltpu.sync_copy(src_hbm.at[pl.ds(base, CH)], idx_s)
            pltpu.sync_copy(dst_hbm.at[pl.ds(base, CH)], idx_d)
            ca = pltpu.async_copy(x_hbm.at[idx_s], rows_s, sem_s)
            cb = pltpu.async_copy(x_hbm.at[idx_d], rows_d, sem_d)
            ca.wait()
            cb.wait()
            pltpu.sync_copy(rows_s, xs_out.at[pl.ds(base, CH)])
            pltpu.sync_copy(rows_d, xd_out.at[pl.ds(base, CH)])

        return carry

    lax.fori_loop(jnp.int32(0), jnp.int32(TRIPS), body, None)


@functools.partial(
    pl.kernel,
    out_type=jax.ShapeDtypeStruct((2 * NP, WIDTH), jnp.float32),
    mesh=_mesh,
    scratch_types=[
        pltpu.VMEM_SHARED((NP, WIDTH), jnp.float32),
        pltpu.VMEM((CH,), jnp.int32),
        pltpu.VMEM((CH, WIDTH), jnp.float32),
    ],
)
def _sc_scatter(dst_hbm, msgp_hbm, zero_hbm, out_hbm, agg_sh, idx, upd):
    i32 = jnp.int32
    cid = lax.axis_index("c")
    sid = lax.axis_index("s")
    wid = sid * i32(2) + cid
    rbase = sid * i32(ROWS_PT)

    pltpu.sync_copy(zero_hbm.at[pl.ds(rbase, ROWS_PT)],
                    agg_sh.at[pl.ds(rbase, ROWS_PT)])
    plsc.subcore_barrier()

    def body(k, carry):
        c = wid + k * i32(NW)

        @pl.when(c < i32(NCH))
        def _go():
            base = c * i32(CH)
            pltpu.sync_copy(dst_hbm.at[pl.ds(base, CH)], idx)
            pltpu.sync_copy(msgp_hbm.at[pl.ds(base, CH)], upd)
            pltpu.sync_copy(upd, agg_sh.at[idx], add=True)

        return carry

    lax.fori_loop(jnp.int32(0), jnp.int32(TRIPS), body, None)
    plsc.subcore_barrier()
    pltpu.sync_copy(agg_sh.at[pl.ds(rbase, ROWS_PT)],
                    out_hbm.at[pl.ds(cid * i32(NP) + rbase, ROWS_PT)])


def _edge_block(xs_ref, xd_ref, ea_ref, w0t, w1t, bsum, gt, emb, bg, bv, wpt,
                out_ref):
    f32 = jnp.float32
    xx = (jnp.dot(xs_ref[...], w0t[...], preferred_element_type=f32)
          + jnp.dot(xd_ref[...], w1t[...], preferred_element_type=f32)
          + bsum[...])
    gtm = gt[...]
    dn = (((1,), (1,)), ((), ()))
    s1 = lax.dot_general(xx, gtm, dn, preferred_element_type=f32)
    s2 = lax.dot_general(xx * xx, gtm, dn, preferred_element_type=f32)
    mu = s1 * (1.0 / 32.0)
    var = s2 * (1.0 / 32.0) - mu * mu
    rstd = lax.rsqrt(var + EPS)
    xxn = (xx - jnp.dot(mu, gtm, preferred_element_type=f32)) \
        * jnp.dot(rstd, gtm, preferred_element_type=f32)
    ea = ea_ref[...]
    io = lax.broadcasted_iota(jnp.int32, (EB, BOND), 1)
    cnts = jnp.zeros((EB, BOND), f32)
    for l in range(L):
        cnts = cnts + (ea[:, l:l + 1] == io).astype(f32)
    cnts = cnts * (io > 0).astype(f32)
    tot = jnp.sum(cnts, axis=1, keepdims=True)
    bag = jnp.dot(cnts, emb[...], preferred_element_type=f32) \
        / jnp.maximum(tot, 1.0)
    pre_gate = xxn + bag
    gate = jnp.maximum(jnp.dot(pre_gate, bg[...], preferred_element_type=f32),
                       0.0)
    val = jnp.dot(xxn, bv[...], preferred_element_type=f32)
    out_ref[...] = jnp.dot(gate * val, wpt[...], preferred_element_type=f32)


def _finish_block(aggA, aggB, deg_ref, bp, dp, out_ref):
    scale = jnp.exp(dp[...] * jnp.log(deg_ref[...]))
    out_ref[...] = scale * (aggA[...] + aggB[...] + bp[...])


_Z = np.int32(0)


def _full(shape):
    return pl.BlockSpec(shape, lambda i: (_Z,) * len(shape))


def kernel(x, deg, edge_idx, edge_attr, W_pre0, b_pre0, W_pre1, b_pre1,
           emb_table, W_gate, W_value, W_post, b_post, degree_param):
    f32 = jnp.float32
    x = x.astype(f32)
    src = edge_idx[0].astype(jnp.int32)
    dst = edge_idx[1].astype(jnp.int32)
    ea = edge_attr.astype(jnp.int32)

    w0t = W_pre0.T.astype(f32)
    w1t = W_pre1.T.astype(f32)
    bsum = (b_pre0 + b_pre1).reshape(1, C1).astype(f32)
    gt = (jnp.arange(C1) // (C1 // NUM_HEAD)
          == jnp.arange(NUM_HEAD)[:, None]).astype(f32)
    pg = C1 // NUM_HEAD
    bg = jnp.zeros((C1, CA), f32)
    bv = jnp.zeros((C1, CA), f32)
    for g in range(NUM_HEAD):
        sl = slice(g * pg, (g + 1) * pg)
        bg = bg.at[sl, sl].set(W_gate[g].T.astype(f32))
        bv = bv.at[sl, sl].set(W_value[g].T.astype(f32))
    wpt = W_post.T.astype(f32)

    xs, xd = _sc_gather(src, dst, x)

    grid = E // EB
    msgp = pl.pallas_call(
        _edge_block,
        grid=(grid,),
        in_specs=[
            pl.BlockSpec((EB, WIDTH), lambda i: (i, _Z)),
            pl.BlockSpec((EB, WIDTH), lambda i: (i, _Z)),
            pl.BlockSpec((EB, L), lambda i: (i, _Z)),
            _full((WIDTH, C1)),
            _full((WIDTH, C1)),
            _full((1, C1)),
            _full((NUM_HEAD, C1)),
            _full((BOND, C1)),
            _full((C1, CA)),
            _full((C1, CA)),
            _full((CA, WIDTH)),
        ],
        out_specs=pl.BlockSpec((EB, WIDTH), lambda i: (i, _Z)),
        out_shape=jax.ShapeDtypeStruct((E, WIDTH), f32),
    )(xs, xd, ea, w0t, w1t, bsum, gt, emb_table.astype(f32), bg, bv, wpt)

    zero = jnp.zeros((NP, WIDTH), f32)
    agg2 = _sc_scatter(dst, msgp, zero)

    out = pl.pallas_call(
        _finish_block,
        grid=(1,),
        in_specs=[
            _full((N, WIDTH)),
            _full((N, WIDTH)),
            _full((N, 1)),
            _full((1, WIDTH)),
            _full((1, WIDTH)),
        ],
        out_specs=_full((N, WIDTH)),
        out_shape=jax.ShapeDtypeStruct((N, WIDTH), f32),
    )(agg2[:N], agg2[NP:NP + N], deg.reshape(N, 1).astype(f32),
      b_post.reshape(1, WIDTH).astype(f32),
      degree_param.reshape(1, WIDTH).astype(f32))
    return out.astype(jnp.float64)


# speedup vs baseline: 208.0341x; 2.0870x over previous
import functools

import jax
import jax.numpy as jnp
import numpy as np
from jax import lax
from jax.experimental import pallas as pl
from jax.experimental.pallas import tpu as pltpu
from jax.experimental.pallas import tpu_sc as plsc

N = 10000
E = 320000
WIDTH = 128
C1 = 256
CA = 256
L = 4
BOND = 64
NUM_HEAD = 8
EPS = 1e-5

CH = 128
NCH = E // CH
NW = 32
TRIPS = -(-NCH // NW)
EB = 2560
NP = 10240
ROWS_PT = NP // 16

_mesh = plsc.VectorSubcoreMesh(core_axis_name="c", subcore_axis_name="s")


@functools.partial(
    pl.kernel,
    out_type=(jax.ShapeDtypeStruct((E, WIDTH), jnp.float32),
              jax.ShapeDtypeStruct((E, WIDTH), jnp.float32)),
    mesh=_mesh,
    scratch_types=[
        pltpu.VMEM((CH,), jnp.int32),
        pltpu.VMEM((CH,), jnp.int32),
        pltpu.VMEM((CH, WIDTH), jnp.float32),
        pltpu.VMEM((CH, WIDTH), jnp.float32),
        pltpu.SemaphoreType.DMA,
        pltpu.SemaphoreType.DMA,
    ],
)
def _sc_gather(src_hbm, dst_hbm, x_hbm, xs_out, xd_out,
               idx_s, idx_d, rows_s, rows_d, sem_s, sem_d):
    i32 = jnp.int32
    cid = lax.axis_index("c")
    sid = lax.axis_index("s")
    wid = sid * i32(2) + cid

    def body(k, carry):
        c = wid + k * i32(NW)

        @pl.when(c < i32(NCH))
        def _go():
            base = c * i32(CH)
            pltpu.sync_copy(src_hbm.at[pl.ds(base, CH)], idx_s)
            pltpu.sync_copy(dst_hbm.at[pl.ds(base, CH)], idx_d)
            ca = pltpu.async_copy(x_hbm.at[idx_s], rows_s, sem_s)
            cb = pltpu.async_copy(x_hbm.at[idx_d], rows_d, sem_d)
            ca.wait()
            cb.wait()
            pltpu.sync_copy(rows_s, xs_out.at[pl.ds(base, CH)])
            pltpu.sync_copy(rows_d, xd_out.at[pl.ds(base, CH)])

        return carry

    lax.fori_loop(jnp.int32(0), jnp.int32(TRIPS), body, None)


@functools.partial(
    pl.kernel,
    out_type=jax.ShapeDtypeStruct((2 * NP, WIDTH), jnp.float32),
    mesh=_mesh,
    scratch_types=[
        pltpu.VMEM_SHARED((NP, WIDTH), jnp.float32),
        pltpu.VMEM((CH,), jnp.int32),
        pltpu.VMEM((CH, WIDTH), jnp.float32),
    ],
)
def _sc_scatter(dst_hbm, msgp_hbm, zero_hbm, out_hbm, agg_sh, idx, upd):
    i32 = jnp.int32
    cid = lax.axis_index("c")
    sid = lax.axis_index("s")
    wid = sid * i32(2) + cid
    rbase = sid * i32(ROWS_PT)

    pltpu.sync_copy(zero_hbm.at[pl.ds(rbase, ROWS_PT)],
                    agg_sh.at[pl.ds(rbase, ROWS_PT)])
    plsc.subcore_barrier()

    def body(k, carry):
        c = wid + k * i32(NW)

        @pl.when(c < i32(NCH))
        def _go():
            base = c * i32(CH)
            pltpu.sync_copy(dst_hbm.at[pl.ds(base, CH)], idx)
            pltpu.sync_copy(msgp_hbm.at[pl.ds(base, CH)], upd)
            pltpu.sync_copy(upd, agg_sh.at[idx], add=True)

        return carry

    lax.fori_loop(jnp.int32(0), jnp.int32(TRIPS), body, None)
    plsc.subcore_barrier()
    pltpu.sync_copy(agg_sh.at[pl.ds(rbase, ROWS_PT)],
                    out_hbm.at[pl.ds(cid * i32(NP) + rbase, ROWS_PT)])


def _edge_block(xs_ref, xd_ref, ea_ref, w0t, w1t, bsum, gt, emb, ones8, bg,
                bv, wpt, out_ref):
    f32 = jnp.float32
    xx = (jnp.dot(xs_ref[...], w0t[...], preferred_element_type=f32)
          + jnp.dot(xd_ref[...], w1t[...], preferred_element_type=f32)
          + bsum[...])
    gtm = gt[...]
    dn = (((1,), (1,)), ((), ()))
    s1 = lax.dot_general(xx, gtm, dn, preferred_element_type=f32)
    s2 = lax.dot_general(xx * xx, gtm, dn, preferred_element_type=f32)
    mu = s1 * (1.0 / 32.0)
    var = s2 * (1.0 / 32.0) - mu * mu
    rstd = lax.rsqrt(var + EPS)
    xxn = (xx - jnp.dot(mu, gtm, preferred_element_type=f32)) \
        * jnp.dot(rstd, gtm, preferred_element_type=f32)
    ea = ea_ref[...]
    io = lax.broadcasted_iota(jnp.int32, (BOND, EB), 0)
    cnts = jnp.zeros((BOND, EB), f32)
    for l in range(L):
        cnts = cnts + (ea[l:l + 1, :] == io).astype(f32)
    cnts = cnts * (io > 0).astype(f32)
    dn0 = (((0,), (0,)), ((), ()))
    bagsum = lax.dot_general(cnts, emb[...], dn0,
                             preferred_element_type=f32)
    tot = lax.dot_general(cnts, ones8[...], dn0,
                          preferred_element_type=f32)[:, 0:1]
    bag = bagsum / jnp.maximum(tot, 1.0)
    pre_gate = xxn + bag
    gate = jnp.maximum(jnp.dot(pre_gate, bg[...], preferred_element_type=f32),
                       0.0)
    val = jnp.dot(xxn, bv[...], preferred_element_type=f32)
    out_ref[...] = jnp.dot(gate * val, wpt[...], preferred_element_type=f32)


def _finish_block(aggA, aggB, deg_ref, bp, dp, out_ref):
    scale = jnp.exp(dp[...] * jnp.log(deg_ref[...]))
    out_ref[...] = scale * (aggA[...] + aggB[...] + bp[...])


_Z = np.int32(0)


def _full(shape):
    return pl.BlockSpec(shape, lambda i: (_Z,) * len(shape))


def kernel(x, deg, edge_idx, edge_attr, W_pre0, b_pre0, W_pre1, b_pre1,
           emb_table, W_gate, W_value, W_post, b_post, degree_param):
    f32 = jnp.float32
    x = x.astype(f32)
    eidx = edge_idx.astype(jnp.int32)
    src = eidx[0]
    dst = eidx[1]
    ea = edge_attr.T.astype(jnp.int32)

    w0t = W_pre0.T.astype(f32)
    w1t = W_pre1.T.astype(f32)
    bsum = (b_pre0 + b_pre1).reshape(1, C1).astype(f32)
    gt = (jnp.arange(C1) // (C1 // NUM_HEAD)
          == jnp.arange(NUM_HEAD)[:, None]).astype(f32)
    pg = C1 // NUM_HEAD
    bg = jnp.zeros((C1, CA), f32)
    bv = jnp.zeros((C1, CA), f32)
    for g in range(NUM_HEAD):
        sl = slice(g * pg, (g + 1) * pg)
        bg = bg.at[sl, sl].set(W_gate[g].T.astype(f32))
        bv = bv.at[sl, sl].set(W_value[g].T.astype(f32))
    wpt = W_post.T.astype(f32)

    xs, xd = _sc_gather(src, dst, x)

    grid = E // EB
    msgp = pl.pallas_call(
        _edge_block,
        grid=(grid,),
        in_specs=[
            pl.BlockSpec((EB, WIDTH), lambda i: (i, _Z)),
            pl.BlockSpec((EB, WIDTH), lambda i: (i, _Z)),
            pl.BlockSpec((L, EB), lambda i: (_Z, i)),
            _full((WIDTH, C1)),
            _full((WIDTH, C1)),
            _full((1, C1)),
            _full((NUM_HEAD, C1)),
            _full((BOND, C1)),
            _full((BOND, 8)),
            _full((C1, CA)),
            _full((C1, CA)),
            _full((CA, WIDTH)),
        ],
        out_specs=pl.BlockSpec((EB, WIDTH), lambda i: (i, _Z)),
        out_shape=jax.ShapeDtypeStruct((E, WIDTH), f32),
    )(xs, xd, ea, w0t, w1t, bsum, gt, emb_table.astype(f32),
      jnp.ones((BOND, 8), f32), bg, bv, wpt)

    zero = jnp.zeros((NP, WIDTH), f32)
    agg2 = _sc_scatter(dst, msgp, zero)

    out = pl.pallas_call(
        _finish_block,
        grid=(1,),
        in_specs=[
            _full((N, WIDTH)),
            _full((N, WIDTH)),
            _full((N, 1)),
            _full((1, WIDTH)),
            _full((1, WIDTH)),
        ],
        out_specs=_full((N, WIDTH)),
        out_shape=jax.ShapeDtypeStruct((N, WIDTH), f32),
    )(agg2[:N], agg2[NP:NP + N], deg.reshape(N, 1).astype(f32),
      b_post.reshape(1, WIDTH).astype(f32),
      degree_param.reshape(1, WIDTH).astype(f32))
    return out.astype(jnp.float64)


# speedup vs baseline: 213.8486x; 1.0280x over previous
import functools

import jax
import jax.numpy as jnp
import numpy as np
from jax import lax
from jax.experimental import pallas as pl
from jax.experimental.pallas import tpu as pltpu
from jax.experimental.pallas import tpu_sc as plsc

N = 10000
E = 320000
WIDTH = 128
C1 = 256
CA = 256
L = 4
BOND = 64
NUM_HEAD = 8
EPS = 1e-5

CH = 128
NCH = E // CH
NW = 32
TRIPS = -(-NCH // NW)
EB = 2560
NP = 10240
ROWS_PT = NP // 16

_mesh = plsc.VectorSubcoreMesh(core_axis_name="c", subcore_axis_name="s")


@functools.partial(
    pl.kernel,
    out_type=(jax.ShapeDtypeStruct((E, WIDTH), jnp.float32),
              jax.ShapeDtypeStruct((E, WIDTH), jnp.float32)),
    mesh=_mesh,
    scratch_types=[
        pltpu.VMEM((CH,), jnp.int32),
        pltpu.VMEM((CH,), jnp.int32),
        pltpu.VMEM((CH, WIDTH), jnp.float32),
        pltpu.VMEM((CH, WIDTH), jnp.float32),
        pltpu.SemaphoreType.DMA,
        pltpu.SemaphoreType.DMA,
    ],
)
def _sc_gather(src_hbm, dst_hbm, x_hbm, xs_out, xd_out,
               idx_s, idx_d, rows_s, rows_d, sem_s, sem_d):
    i32 = jnp.int32
    cid = lax.axis_index("c")
    sid = lax.axis_index("s")
    wid = sid * i32(2) + cid

    def body(k, carry):
        c = wid + k * i32(NW)

        @pl.when(c < i32(NCH))
        def _go():
            base = c * i32(CH)
            pltpu.sync_copy(src_hbm.at[pl.ds(base, CH)], idx_s)
            pltpu.sync_copy(dst_hbm.at[pl.ds(base, CH)], idx_d)
            ca = pltpu.async_copy(x_hbm.at[idx_s], rows_s, sem_s)
            cb = pltpu.async_copy(x_hbm.at[idx_d], rows_d, sem_d)
            ca.wait()
            cb.wait()
            pltpu.sync_copy(rows_s, xs_out.at[pl.ds(base, CH)])
            pltpu.sync_copy(rows_d, xd_out.at[pl.ds(base, CH)])

        return carry

    lax.fori_loop(jnp.int32(0), jnp.int32(TRIPS), body, None)


@functools.partial(
    pl.kernel,
    out_type=jax.ShapeDtypeStruct((2 * NP, WIDTH), jnp.float32),
    mesh=_mesh,
    scratch_types=[
        pltpu.VMEM_SHARED((NP, WIDTH), jnp.float32),
        pltpu.VMEM((CH,), jnp.int32),
        pltpu.VMEM((CH, WIDTH), jnp.float32),
    ],
)
def _sc_scatter(dst_hbm, msgp_hbm, zero_hbm, out_hbm, agg_sh, idx, upd):
    i32 = jnp.int32
    cid = lax.axis_index("c")
    sid = lax.axis_index("s")
    wid = sid * i32(2) + cid
    rbase = sid * i32(ROWS_PT)

    pltpu.sync_copy(zero_hbm.at[pl.ds(rbase, ROWS_PT)],
                    agg_sh.at[pl.ds(rbase, ROWS_PT)])
    plsc.subcore_barrier()

    def body(k, carry):
        c = wid + k * i32(NW)

        @pl.when(c < i32(NCH))
        def _go():
            base = c * i32(CH)
            pltpu.sync_copy(dst_hbm.at[pl.ds(base, CH)], idx)
            pltpu.sync_copy(msgp_hbm.at[pl.ds(base, CH)], upd)
            pltpu.sync_copy(upd, agg_sh.at[idx], add=True)

        return carry

    lax.fori_loop(jnp.int32(0), jnp.int32(TRIPS), body, None)
    plsc.subcore_barrier()
    pltpu.sync_copy(agg_sh.at[pl.ds(rbase, ROWS_PT)],
                    out_hbm.at[pl.ds(cid * i32(NP) + rbase, ROWS_PT)])


def _edge_block(xs_ref, xd_ref, ea_ref, w0t, w1t, bsum, gt, emb, bg,
                bv, wpt, out_ref):
    f32 = jnp.float32
    xx = (jnp.dot(xs_ref[...], w0t[...], preferred_element_type=f32)
          + jnp.dot(xd_ref[...], w1t[...], preferred_element_type=f32)
          + bsum[...])
    gtm = gt[...]
    dn = (((1,), (1,)), ((), ()))
    s1 = lax.dot_general(xx, gtm, dn, preferred_element_type=f32)
    s2 = lax.dot_general(xx * xx, gtm, dn, preferred_element_type=f32)
    mu = s1 * (1.0 / 32.0)
    var = s2 * (1.0 / 32.0) - mu * mu
    rstd = lax.rsqrt(var + EPS)
    xxn = (xx - jnp.dot(mu, gtm, preferred_element_type=f32)) \
        * jnp.dot(rstd, gtm, preferred_element_type=f32)
    ea = ea_ref[...]
    io = lax.broadcasted_iota(jnp.int32, (BOND, EB), 0)
    cnts = jnp.zeros((BOND, EB), f32)
    tot = jnp.zeros((1, EB), f32)
    for l in range(L):
        al = ea[l:l + 1, :]
        cnts = cnts + (al == io).astype(f32)
        tot = tot + (al != 0).astype(f32)
    cnts = cnts * (1.0 / jnp.maximum(tot, 1.0))
    dn0 = (((0,), (0,)), ((), ()))
    bag = lax.dot_general(cnts, emb[...], dn0,
                          preferred_element_type=f32)
    pre_gate = xxn + bag
    gate = jnp.maximum(jnp.dot(pre_gate, bg[...], preferred_element_type=f32),
                       0.0)
    val = jnp.dot(xxn, bv[...], preferred_element_type=f32)
    out_ref[...] = jnp.dot(gate * val, wpt[...], preferred_element_type=f32)


def _finish_block(aggA, aggB, deg_ref, bp, dp, out_ref):
    scale = jnp.exp(dp[...] * jnp.log(deg_ref[...]))
    out_ref[...] = scale * (aggA[...] + aggB[...] + bp[...])


_Z = np.int32(0)


def _full(shape):
    return pl.BlockSpec(shape, lambda i: (_Z,) * len(shape))


def kernel(x, deg, edge_idx, edge_attr, W_pre0, b_pre0, W_pre1, b_pre1,
           emb_table, W_gate, W_value, W_post, b_post, degree_param):
    f32 = jnp.float32
    x = x.astype(f32)
    eidx = edge_idx.astype(jnp.int32)
    src = eidx[0]
    dst = eidx[1]
    ea = edge_attr.T.astype(jnp.int32)

    w0t = W_pre0.T.astype(f32)
    w1t = W_pre1.T.astype(f32)
    bsum = (b_pre0 + b_pre1).reshape(1, C1).astype(f32)
    gt = (jnp.arange(C1) // (C1 // NUM_HEAD)
          == jnp.arange(NUM_HEAD)[:, None]).astype(f32)
    pg = C1 // NUM_HEAD
    bg = jnp.zeros((C1, CA), f32)
    bv = jnp.zeros((C1, CA), f32)
    for g in range(NUM_HEAD):
        sl = slice(g * pg, (g + 1) * pg)
        bg = bg.at[sl, sl].set(W_gate[g].T.astype(f32))
        bv = bv.at[sl, sl].set(W_value[g].T.astype(f32))
    wpt = W_post.T.astype(f32)

    xs, xd = _sc_gather(src, dst, x)

    grid = E // EB
    msgp = pl.pallas_call(
        _edge_block,
        grid=(grid,),
        in_specs=[
            pl.BlockSpec((EB, WIDTH), lambda i: (i, _Z)),
            pl.BlockSpec((EB, WIDTH), lambda i: (i, _Z)),
            pl.BlockSpec((L, EB), lambda i: (_Z, i)),
            _full((WIDTH, C1)),
            _full((WIDTH, C1)),
            _full((1, C1)),
            _full((NUM_HEAD, C1)),
            _full((BOND, C1)),
            _full((C1, CA)),
            _full((C1, CA)),
            _full((CA, WIDTH)),
        ],
        out_specs=pl.BlockSpec((EB, WIDTH), lambda i: (i, _Z)),
        out_shape=jax.ShapeDtypeStruct((E, WIDTH), f32),
    )(xs, xd, ea, w0t, w1t, bsum, gt, emb_table.astype(f32), bg, bv, wpt)

    zero = jnp.zeros((NP, WIDTH), f32)
    agg2 = _sc_scatter(dst, msgp, zero)

    out = pl.pallas_call(
        _finish_block,
        grid=(1,),
        in_specs=[
            _full((N, WIDTH)),
            _full((N, WIDTH)),
            _full((N, 1)),
            _full((1, WIDTH)),
            _full((1, WIDTH)),
        ],
        out_specs=_full((N, WIDTH)),
        out_shape=jax.ShapeDtypeStruct((N, WIDTH), f32),
    )(agg2[:N], agg2[NP:NP + N], deg.reshape(N, 1).astype(f32),
      b_post.reshape(1, WIDTH).astype(f32),
      degree_param.reshape(1, WIDTH).astype(f32))
    return out.astype(jnp.float64)
